# Initial kernel scaffold; baseline (speedup 1.0000x reference)
import jax
import jax.numpy as jnp
from jax import lax
from jax.experimental import pallas as pl
from jax.experimental.pallas import tpu as pltpu
from jax.experimental.pallas import tpu_sc as plsc

N = 50000
E = 800000
H = 64
D = 16
L = 4
G = 128

NC = 2
NS = 16
HH = H // 2
EB = 80
E_PER_TILE = E // NS
NBLK = E_PER_TILE // EB
N_PER_TILE = N // NS


def _sigmoid(v):
    return 1.0 / (1.0 + jnp.exp(-v))


def _softplus16(v):
    t = jnp.exp(-jnp.abs(v))
    g = t * (1.0 + t * (-0.5 + t * (0.3333333 + t * (-0.25 + t * 0.2))))
    g = g - 1.0 + (1.0 + t) * jnp.exp(-g)
    return g + jnp.maximum(v, 0.0)


def _edge_kernel_body(td_hbm, ts_hbm, c_hbm, dst_hbm, src_hbm, zero_hbm,
                      out_hbm, idx_d, idx_s, idx_dg, idx_sg, rows_d, rows_s,
                      rows_c, msg, acc, sem):
    c = lax.axis_index("c")
    s = lax.axis_index("s")
    cN = c * N

    pltpu.sync_copy(zero_hbm.at[pl.ds(s * N_PER_TILE, N_PER_TILE)],
                    acc.at[pl.ds(s * N_PER_TILE, N_PER_TILE)])
    plsc.subcore_barrier()

    def block(b, carry):
        e0 = s * E_PER_TILE + b * EB
        pltpu.sync_copy(dst_hbm.at[pl.ds(e0, EB)], idx_d)
        pltpu.sync_copy(src_hbm.at[pl.ds(e0, EB)], idx_s)
        for i in range(EB // 16):
            sl = pl.ds(i * 16, 16)
            idx_dg[sl] = idx_d[sl] + cN
            idx_sg[sl] = idx_s[sl] + cN
        gd = pltpu.async_copy(td_hbm.at[idx_dg], rows_d, sem)
        gs = pltpu.async_copy(ts_hbm.at[idx_sg], rows_s, sem)
        pltpu.sync_copy(c_hbm.at[pl.ds(c * E + e0, EB)], rows_c)
        gd.wait()
        gs.wait()

        def edge(i, carry2):
            f0 = rows_d[i, pl.ds(0, 16)] + rows_s[i, pl.ds(0, 16)] + rows_c[i, pl.ds(0, 16)]
            f1 = rows_d[i, pl.ds(16, 16)] + rows_s[i, pl.ds(16, 16)] + rows_c[i, pl.ds(16, 16)]
            s0 = rows_d[i, pl.ds(32, 16)] + rows_s[i, pl.ds(32, 16)] + rows_c[i, pl.ds(32, 16)]
            s1 = rows_d[i, pl.ds(48, 16)] + rows_s[i, pl.ds(48, 16)] + rows_c[i, pl.ds(48, 16)]
            msg[i, pl.ds(0, 16)] = _sigmoid(f0) * _softplus16(s0)
            msg[i, pl.ds(16, 16)] = _sigmoid(f1) * _softplus16(s1)
            return carry2

        lax.fori_loop(0, EB, edge, 0)
        pltpu.sync_copy(msg, acc.at[idx_d], add=True)
        return carry

    lax.fori_loop(0, NBLK, block, 0)
    plsc.subcore_barrier()
    pltpu.sync_copy(acc.at[pl.ds(s * N_PER_TILE, N_PER_TILE)],
                    out_hbm.at[c].at[pl.ds(s * N_PER_TILE, N_PER_TILE)])


@jax.jit
def _edge_pass(td, ts, cc, dst, src, zero):
    k = pl.kernel(
        _edge_kernel_body,
        out_type=jax.ShapeDtypeStruct((NC, N, HH), jnp.float32),
        mesh=plsc.VectorSubcoreMesh(core_axis_name="c", subcore_axis_name="s"),
        scratch_types=[
            pltpu.VMEM((EB,), jnp.int32),
            pltpu.VMEM((EB,), jnp.int32),
            pltpu.VMEM((EB,), jnp.int32),
            pltpu.VMEM((EB,), jnp.int32),
            pltpu.VMEM((EB, 2 * HH), jnp.float32),
            pltpu.VMEM((EB, 2 * HH), jnp.float32),
            pltpu.VMEM((EB, 2 * HH), jnp.float32),
            pltpu.VMEM((EB, HH), jnp.float32),
            pltpu.VMEM_SHARED((N, HH), jnp.float32),
            pltpu.SemaphoreType.DMA,
        ],
    )
    return k(td, ts, cc, dst, src, zero)


def _batchnorm(h, g, b):
    m = h.mean(axis=0)
    v = h.var(axis=0)
    return (h - m) / jnp.sqrt(v + 1e-5) * g + b


def kernel(z, edge_index, edge_attr, batch, emb, Wf, bf, Ws, bs,
           bn1_g, bn1_b, bn2_g, bn2_b, W1, b1, W2, b2):
    x = emb[z]
    src = edge_index[0].astype(jnp.int32)
    dst = edge_index[1].astype(jnp.int32)
    zero = jnp.zeros((N, HH), jnp.float32)

    for l in range(L):
        Wfl, Wsl = Wf[l], Ws[l]
        tf_d = x @ Wfl[:H]
        tf_s = x @ Wfl[H:2 * H]
        ts_d = x @ Wsl[:H]
        ts_s = x @ Wsl[H:2 * H]
        td = jnp.concatenate([
            jnp.concatenate([tf_d[:, :HH], ts_d[:, :HH]], axis=1),
            jnp.concatenate([tf_d[:, HH:], ts_d[:, HH:]], axis=1)], axis=0)
        tsm = jnp.concatenate([
            jnp.concatenate([tf_s[:, :HH], ts_s[:, :HH]], axis=1),
            jnp.concatenate([tf_s[:, HH:], ts_s[:, HH:]], axis=1)], axis=0)
        cf = edge_attr @ Wfl[2 * H:] + bf[l]
        cs = edge_attr @ Wsl[2 * H:] + bs[l]
        cc = jnp.concatenate([
            jnp.concatenate([cf[:, :HH], cs[:, :HH]], axis=1),
            jnp.concatenate([cf[:, HH:], cs[:, HH:]], axis=1)], axis=0)

        out2 = _edge_pass(td, tsm, cc, dst, src, zero)
        agg = jnp.concatenate([out2[0], out2[1]], axis=1)

        x = _batchnorm(agg, bn1_g[l], bn1_b[l]) + x
        x = _batchnorm(x, bn2_g[l], bn2_b[l])
        x = jax.nn.softplus(x)

    sums = jax.ops.segment_sum(x, batch, num_segments=G)
    cnt = jax.ops.segment_sum(jnp.ones((N, 1), dtype=x.dtype), batch, num_segments=G)
    pooled = sums / jnp.maximum(cnt, 1.0)
    h = jax.nn.silu(pooled @ W1 + b1)
    out = (h @ W2 + b2).squeeze(-1)
    return out


# speedup vs baseline: 1.0994x; 1.0994x over previous
import jax
import jax.numpy as jnp
from jax import lax
from jax.experimental import pallas as pl
from jax.experimental.pallas import tpu as pltpu
from jax.experimental.pallas import tpu_sc as plsc

N = 50000
E = 800000
H = 64
D = 16
L = 4
G = 128

NC = 2
NS = 16
HH = H // 2
EB = 80
E_PER_TILE = E // NS
NBLK = E_PER_TILE // EB
N_PER_TILE = N // NS


def _sigmoid(v):
    return 1.0 / (1.0 + jnp.exp(-v))


def _softplus16(v):
    t = jnp.exp(-jnp.abs(v))
    g = t * (1.0 + t * (-0.5 + t * (0.3333333 + t * (-0.25 + t * 0.2))))
    g = g - 1.0 + (1.0 + t) * jnp.exp(-g)
    return g + jnp.maximum(v, 0.0)


def _edge_kernel_body(td_hbm, ts_hbm, c_hbm, dst_hbm, src_hbm, zero_hbm,
                      out_hbm, idx_d, idx_s, idx_dg, idx_sg, rows_d, rows_s,
                      rows_c, msg, acc, sem):
    c = lax.axis_index("c")
    s = lax.axis_index("s")
    cN = c * N

    pltpu.sync_copy(zero_hbm.at[pl.ds(s * N_PER_TILE, N_PER_TILE)],
                    acc.at[pl.ds(s * N_PER_TILE, N_PER_TILE)])
    plsc.subcore_barrier()

    def block(b, carry):
        e0 = s * E_PER_TILE + b * EB
        pltpu.sync_copy(dst_hbm.at[pl.ds(e0, EB)], idx_d)
        pltpu.sync_copy(src_hbm.at[pl.ds(e0, EB)], idx_s)
        for i in range(EB // 16):
            sl = pl.ds(i * 16, 16)
            idx_dg[sl] = idx_d[sl] + cN
            idx_sg[sl] = idx_s[sl] + cN
        gd = pltpu.async_copy(td_hbm.at[idx_dg], rows_d, sem)
        gs = pltpu.async_copy(ts_hbm.at[idx_sg], rows_s, sem)
        pltpu.sync_copy(c_hbm.at[pl.ds(c * E + e0, EB)], rows_c)
        gd.wait()
        gs.wait()

        def edge(i, carry2):
            f0 = rows_d[i, pl.ds(0, 16)] + rows_s[i, pl.ds(0, 16)] + rows_c[i, pl.ds(0, 16)]
            f1 = rows_d[i, pl.ds(16, 16)] + rows_s[i, pl.ds(16, 16)] + rows_c[i, pl.ds(16, 16)]
            s0 = rows_d[i, pl.ds(32, 16)] + rows_s[i, pl.ds(32, 16)] + rows_c[i, pl.ds(32, 16)]
            s1 = rows_d[i, pl.ds(48, 16)] + rows_s[i, pl.ds(48, 16)] + rows_c[i, pl.ds(48, 16)]
            msg[i, pl.ds(0, 16)] = _sigmoid(f0) * _softplus16(s0)
            msg[i, pl.ds(16, 16)] = _sigmoid(f1) * _softplus16(s1)
            return carry2

        lax.fori_loop(0, EB, edge, 0)
        pltpu.sync_copy(msg, acc.at[idx_d], add=True)
        return carry

    lax.fori_loop(0, NBLK, block, 0)
    plsc.subcore_barrier()
    pltpu.sync_copy(acc.at[pl.ds(s * N_PER_TILE, N_PER_TILE)],
                    out_hbm.at[c].at[pl.ds(s * N_PER_TILE, N_PER_TILE)])


@jax.jit
def _edge_pass(td, ts, cc, dst, src, zero):
    k = pl.kernel(
        _edge_kernel_body,
        out_type=jax.ShapeDtypeStruct((NC, N, HH), jnp.float32),
        mesh=plsc.VectorSubcoreMesh(core_axis_name="c", subcore_axis_name="s"),
        compiler_params=pltpu.CompilerParams(use_tc_tiling_on_sc=False),
        scratch_types=[
            pltpu.VMEM((EB,), jnp.int32),
            pltpu.VMEM((EB,), jnp.int32),
            pltpu.VMEM((EB,), jnp.int32),
            pltpu.VMEM((EB,), jnp.int32),
            pltpu.VMEM((EB, 2 * HH), jnp.float32),
            pltpu.VMEM((EB, 2 * HH), jnp.float32),
            pltpu.VMEM((EB, 2 * HH), jnp.float32),
            pltpu.VMEM((EB, HH), jnp.float32),
            pltpu.VMEM_SHARED((N, HH), jnp.float32),
            pltpu.SemaphoreType.DMA,
        ],
    )
    return k(td, ts, cc, dst, src, zero)


def _batchnorm(h, g, b):
    m = h.mean(axis=0)
    v = h.var(axis=0)
    return (h - m) / jnp.sqrt(v + 1e-5) * g + b


def kernel(z, edge_index, edge_attr, batch, emb, Wf, bf, Ws, bs,
           bn1_g, bn1_b, bn2_g, bn2_b, W1, b1, W2, b2):
    x = emb[z]
    src = edge_index[0].astype(jnp.int32)
    dst = edge_index[1].astype(jnp.int32)
    zero = jnp.zeros((N, HH), jnp.float32)

    for l in range(L):
        Wfl, Wsl = Wf[l], Ws[l]
        tf_d = x @ Wfl[:H]
        tf_s = x @ Wfl[H:2 * H]
        ts_d = x @ Wsl[:H]
        ts_s = x @ Wsl[H:2 * H]
        td = jnp.concatenate([
            jnp.concatenate([tf_d[:, :HH], ts_d[:, :HH]], axis=1),
            jnp.concatenate([tf_d[:, HH:], ts_d[:, HH:]], axis=1)], axis=0)
        tsm = jnp.concatenate([
            jnp.concatenate([tf_s[:, :HH], ts_s[:, :HH]], axis=1),
            jnp.concatenate([tf_s[:, HH:], ts_s[:, HH:]], axis=1)], axis=0)
        cf = edge_attr @ Wfl[2 * H:] + bf[l]
        cs = edge_attr @ Wsl[2 * H:] + bs[l]
        cc = jnp.concatenate([
            jnp.concatenate([cf[:, :HH], cs[:, :HH]], axis=1),
            jnp.concatenate([cf[:, HH:], cs[:, HH:]], axis=1)], axis=0)

        out2 = _edge_pass(td, tsm, cc, dst, src, zero)
        agg = jnp.concatenate([out2[0], out2[1]], axis=1)

        x = _batchnorm(agg, bn1_g[l], bn1_b[l]) + x
        x = _batchnorm(x, bn2_g[l], bn2_b[l])
        x = jax.nn.softplus(x)

    sums = jax.ops.segment_sum(x, batch, num_segments=G)
    cnt = jax.ops.segment_sum(jnp.ones((N, 1), dtype=x.dtype), batch, num_segments=G)
    pooled = sums / jnp.maximum(cnt, 1.0)
    h = jax.nn.silu(pooled @ W1 + b1)
    out = (h @ W2 + b2).squeeze(-1)
    return out
